# baseline (speedup 1.0000x reference)
import functools

import jax
import jax.numpy as jnp
from jax import lax
from jax.experimental import pallas as pl
from jax.experimental.pallas import tpu as pltpu
from jax.experimental.pallas import tpu_sc as plsc

NN = 10000
EE = 160000
DD = 256
DEE = 16
NPAD = 10240
CHG = 128
NCHG = EE // CHG
CHS = 80
KPT = EE // CHS // 16
RPT = NPAD // 16

_HI = jax.lax.Precision.HIGHEST


def _node_proj_body(x_ref, wes_ref, wed_ref, wma_ref, wmb_ref,
                    pes_ref, ped_ref, pm_ref):
    x = x_ref[...]
    pes_ref[...] = jnp.dot(x, wes_ref[...])
    ped_ref[...] = jnp.dot(x, wed_ref[...])
    pm_ref[0] = jnp.dot(x, wma_ref[...])
    pm_ref[1] = jnp.dot(x, wmb_ref[...])


def _tc_node_proj(x, wes, wed, wma, wmb):
    bn = 2000
    f = jnp.float32
    return pl.pallas_call(
        _node_proj_body,
        grid=(NN // bn,),
        in_specs=[
            pl.BlockSpec((bn, DD), lambda i: (i, 0)),
            pl.BlockSpec((DD, DEE), lambda i: (0, 0)),
            pl.BlockSpec((DD, DEE), lambda i: (0, 0)),
            pl.BlockSpec((DD, 128), lambda i: (0, 0)),
            pl.BlockSpec((DD, 128), lambda i: (0, 0)),
        ],
        out_specs=[
            pl.BlockSpec((bn, DEE), lambda i: (i, 0)),
            pl.BlockSpec((bn, DEE), lambda i: (i, 0)),
            pl.BlockSpec((2, bn, 128), lambda i: (0, i, 0)),
        ],
        out_shape=[
            jax.ShapeDtypeStruct((NN, DEE), f),
            jax.ShapeDtypeStruct((NN, DEE), f),
            jax.ShapeDtypeStruct((2, NN, 128), f),
        ],
    )(x, wes, wed, wma, wmb)


def _pack_t(t, rows):
    ta = jnp.concatenate([t[:, 32 * g:32 * g + 16] for g in range(4)], axis=1)
    tb = jnp.concatenate(
        [t[:, 32 * g + 16:32 * g + 32] for g in range(4)], axis=1)
    ua = jax.lax.bitcast_convert_type(ta, jnp.uint32)
    ub = jax.lax.bitcast_convert_type(tb, jnp.uint32)
    ra = (ua + 0x7FFF + ((ua >> 16) & 1)) >> 16
    rb = (ub + 0x7FFF + ((ub >> 16) & 1)) >> 16
    return jax.lax.bitcast_convert_type(ra | (rb << 16), jnp.int32)


def _edge_dense_body(gsum_ref, ea_ref, wec_ref, be_ref, wma_ref, wmb_ref,
                     bma_ref, bmb_ref, eout_ref, t_ref, *, res):
    ea = ea_ref[...]
    e = jnp.maximum(
        gsum_ref[...] + jnp.dot(ea, wec_ref[...]) + be_ref[...],
        0.0)
    rows = e.shape[0]
    t_ref[0] = _pack_t(jnp.dot(e, wma_ref[...]) + bma_ref[...], rows)
    t_ref[1] = _pack_t(jnp.dot(e, wmb_ref[...]) + bmb_ref[...], rows)
    eout_ref[...] = 0.5 * (ea + e) if res else e


def _tc_edge_dense(gsum, ea, wec, be, wma, wmb, bma, bmb, res):
    be_ = 2000
    f = jnp.float32
    return pl.pallas_call(
        functools.partial(_edge_dense_body, res=res),
        grid=(EE // be_,),
        in_specs=[
            pl.BlockSpec((be_, DEE), lambda i: (i, 0)),
            pl.BlockSpec((be_, DEE), lambda i: (i, 0)),
            pl.BlockSpec((DEE, DEE), lambda i: (0, 0)),
            pl.BlockSpec((1, DEE), lambda i: (0, 0)),
            pl.BlockSpec((DEE, 128), lambda i: (0, 0)),
            pl.BlockSpec((DEE, 128), lambda i: (0, 0)),
            pl.BlockSpec((1, 128), lambda i: (0, 0)),
            pl.BlockSpec((1, 128), lambda i: (0, 0)),
        ],
        out_specs=[
            pl.BlockSpec((be_, DEE), lambda i: (i, 0)),
            pl.BlockSpec((2, be_, 64), lambda i: (0, i, 0)),
        ],
        out_shape=[
            jax.ShapeDtypeStruct((EE, DEE), f),
            jax.ShapeDtypeStruct((2, EE, 64), jnp.int32),
        ],
    )(gsum, ea, wec, be, wma, wmb, bma, bmb)


def _edge_final_body(gsum_ref, ea_ref, wec_ref, be_ref, out_ref):
    out_ref[...] = jnp.maximum(
        gsum_ref[...]
        + jnp.dot(ea_ref[...], wec_ref[...]) + be_ref[...],
        0.0)


def _tc_edge_final(gsum, ea, wec, be):
    be_ = 4000
    return pl.pallas_call(
        _edge_final_body,
        grid=(EE // be_,),
        in_specs=[
            pl.BlockSpec((be_, DEE), lambda i: (i, 0)),
            pl.BlockSpec((be_, DEE), lambda i: (i, 0)),
            pl.BlockSpec((DEE, DEE), lambda i: (0, 0)),
            pl.BlockSpec((1, DEE), lambda i: (0, 0)),
        ],
        out_specs=pl.BlockSpec((be_, DEE), lambda i: (i, 0)),
        out_shape=jax.ShapeDtypeStruct((EE, DEE), jnp.float32),
    )(gsum, ea, wec, be)


def _node_fused_body(x_ref, agga_ref, aggb_ref, wnt_ref, wna_ref, wnb_ref,
                     bn_ref, wes_ref, wed_ref, *rest, res, has_pm):
    if has_pm:
        wma_ref, wmb_ref, xout_ref, pes_ref, ped_ref, pm_ref = rest
    else:
        xout_ref, pes_ref, ped_ref = rest
    x = x_ref[...]
    h = (jnp.dot(x, wnt_ref[...])
         + jnp.dot(agga_ref[0], wna_ref[...])
         + jnp.dot(aggb_ref[0], wnb_ref[...])
         + bn_ref[...])
    h = jnp.maximum(h, 0.0)
    xn = 0.5 * (x + h) if res else h
    xout_ref[...] = xn
    pes_ref[...] = jnp.dot(xn, wes_ref[...])
    ped_ref[...] = jnp.dot(xn, wed_ref[...])
    if has_pm:
        pm_ref[0] = jnp.dot(xn, wma_ref[...])
        pm_ref[1] = jnp.dot(xn, wmb_ref[...])


def _tc_node_fused(x, agg3, wnt, wna, wnb, bn, wes, wed, wma, wmb, res):
    bn_ = 2000
    f = jnp.float32
    has_pm = wma is not None
    in_specs = [
        pl.BlockSpec((bn_, DD), lambda i: (i, 0)),
        pl.BlockSpec((1, bn_, 128), lambda i: (0, i, 0)),
        pl.BlockSpec((1, bn_, 128), lambda i: (1, i, 0)),
        pl.BlockSpec((DD, DD), lambda i: (0, 0)),
        pl.BlockSpec((128, DD), lambda i: (0, 0)),
        pl.BlockSpec((128, DD), lambda i: (0, 0)),
        pl.BlockSpec((1, DD), lambda i: (0, 0)),
        pl.BlockSpec((DD, DEE), lambda i: (0, 0)),
        pl.BlockSpec((DD, DEE), lambda i: (0, 0)),
    ]
    out_specs = [
        pl.BlockSpec((bn_, DD), lambda i: (i, 0)),
        pl.BlockSpec((bn_, DEE), lambda i: (i, 0)),
        pl.BlockSpec((bn_, DEE), lambda i: (i, 0)),
    ]
    out_shape = [
        jax.ShapeDtypeStruct((NN, DD), f),
        jax.ShapeDtypeStruct((NN, DEE), f),
        jax.ShapeDtypeStruct((NN, DEE), f),
    ]
    args = [x, agg3, agg3, wnt, wna, wnb, bn, wes, wed]
    if has_pm:
        in_specs += [pl.BlockSpec((DD, 128), lambda i: (0, 0)),
                     pl.BlockSpec((DD, 128), lambda i: (0, 0))]
        out_specs.append(pl.BlockSpec((2, bn_, 128), lambda i: (0, i, 0)))
        out_shape.append(jax.ShapeDtypeStruct((2, NN, 128), f))
        args += [wma, wmb]
    return pl.pallas_call(
        functools.partial(_node_fused_body, res=res, has_pm=has_pm),
        grid=(NN // bn_,),
        in_specs=in_specs,
        out_specs=out_specs,
        out_shape=out_shape,
    )(*args)


@functools.cache
def _sc_gather_pe_kernel():
    return functools.partial(
        pl.kernel,
        out_type=jax.ShapeDtypeStruct((EE, DEE), jnp.float32),
        mesh=plsc.VectorSubcoreMesh(core_axis_name="c", subcore_axis_name="s"),
        compiler_params=pltpu.CompilerParams(use_tc_tiling_on_sc=False),
        scratch_types=[
            pltpu.VMEM((CHG,), jnp.int32),
            pltpu.VMEM((CHG,), jnp.int32),
            pltpu.VMEM((CHG,), jnp.int32),
            pltpu.VMEM((CHG,), jnp.int32),
            pltpu.VMEM((CHG, DEE), jnp.float32),
            pltpu.VMEM((CHG, DEE), jnp.float32),
            pltpu.VMEM((CHG, DEE), jnp.float32),
            pltpu.VMEM((CHG, DEE), jnp.float32),
            pltpu.SemaphoreType.DMA,
            pltpu.SemaphoreType.DMA,
            pltpu.SemaphoreType.DMA,
            pltpu.SemaphoreType.DMA,
            pltpu.SemaphoreType.DMA,
            pltpu.SemaphoreType.DMA,
        ])(_sc_gather_pe_body)


def _sc_gather_pe(pes, ped, src, dst):
    return _sc_gather_pe_kernel()(pes, ped, src, dst)


def _sc_gather_pe_body(pes_hbm, ped_hbm, src_hbm, dst_hbm, out_hbm,
                       sv0, sv1, dv0, dv1, gs0, gs1, gd0, gd1,
                       semi0, semi1, semg0, semg1, semo0, semo1):
    c = lax.axis_index("c")
    s = lax.axis_index("s")
    w = s * 2 + c
    nk = 39 + jnp.where(w < NCHG - 39 * 32, 1, 0)
    svs, dvs = (sv0, sv1), (dv0, dv1)
    gss, gds = (gs0, gs1), (gd0, gd1)
    semi, semg, semo = (semi0, semi1), (semg0, semg1), (semo0, semo1)

    def e_at(k):
        return (w + 32 * k) * CHG

    def start_idx(k, b):
        e0 = e_at(k)
        pltpu.async_copy(src_hbm.at[pl.ds(e0, CHG)], svs[b], semi[b])
        pltpu.async_copy(dst_hbm.at[pl.ds(e0, CHG)], dvs[b], semi[b])

    def wait_idx(k, b):
        e0 = e_at(k)
        pltpu.make_async_copy(src_hbm.at[pl.ds(e0, CHG)], svs[b], semi[b]).wait()
        pltpu.make_async_copy(dst_hbm.at[pl.ds(e0, CHG)], dvs[b], semi[b]).wait()

    def start_g(k, b):
        pltpu.async_copy(pes_hbm.at[svs[b]], gss[b], semg[b])
        pltpu.async_copy(ped_hbm.at[dvs[b]], gds[b], semg[b])

    def wait_write(k, b):
        e0 = e_at(k)
        pltpu.make_async_copy(gss[b], out_hbm.at[pl.ds(e0, CHG)], semo[b]).wait()

    def finish(k, b):
        e0 = e_at(k)
        pltpu.make_async_copy(pes_hbm.at[svs[b]], gss[b], semg[b]).wait()
        pltpu.make_async_copy(ped_hbm.at[dvs[b]], gds[b], semg[b]).wait()

        def add_body(i, carry):
            gss[b][i, :] = gss[b][i, :] + gds[b][i, :]
            return carry

        lax.fori_loop(0, CHG, add_body, 0, unroll=4)
        pltpu.async_copy(gss[b], out_hbm.at[pl.ds(e0, CHG)], semo[b])

    start_idx(0, 0)
    start_idx(1, 1)
    wait_idx(0, 0)
    start_g(0, 0)

    def body(jj, carry):
        k0 = 2 * jj
        k1 = k0 + 1

        @pl.when(k1 < nk)
        def _():
            wait_idx(k1, 1)

        @pl.when(jnp.logical_and(k1 < nk, k1 >= 2))
        def _():
            wait_write(k1 - 2, 1)

        @pl.when(k1 < nk)
        def _():
            start_g(k1, 1)

        finish(k0, 0)

        @pl.when(k0 + 2 < nk)
        def _():
            start_idx(k0 + 2, 0)

        @pl.when(k1 < nk)
        def _():
            finish(k1, 1)

        @pl.when(k0 + 2 < nk)
        def _():
            wait_idx(k0 + 2, 0)
            wait_write(k0, 0)
            start_g(k0 + 2, 0)

        @pl.when(k1 + 2 < nk)
        def _():
            start_idx(k1 + 2, 1)

        return carry

    lax.fori_loop(0, 20, body, 0)
    wait_write(38, 0)
    wait_write(37, 1)


@functools.cache
def _sc_scatter_kernel():
    return functools.partial(
        pl.kernel,
        out_type=jax.ShapeDtypeStruct((2, NPAD, 128), jnp.float32),
        mesh=plsc.VectorSubcoreMesh(core_axis_name="c", subcore_axis_name="s"),
        compiler_params=pltpu.CompilerParams(
            use_tc_tiling_on_sc=False, needs_layout_passes=False),
        scratch_types=[
            pltpu.VMEM((CHS,), jnp.int32),
            pltpu.VMEM((CHS,), jnp.int32),
            pltpu.VMEM((CHS,), jnp.int32),
            pltpu.VMEM((CHS,), jnp.int32),
            pltpu.VMEM((CHS,), jnp.int32),
            pltpu.VMEM((CHS,), jnp.int32),
            pltpu.VMEM((CHS, 128), jnp.float32),
            pltpu.VMEM((CHS, 128), jnp.float32),
            pltpu.VMEM((CHS, 64), jnp.int32),
            pltpu.VMEM((CHS, 64), jnp.int32),
            pltpu.VMEM_SHARED((NPAD, 128), jnp.float32),
            pltpu.SemaphoreType.DMA,
            pltpu.SemaphoreType.DMA,
            pltpu.SemaphoreType.DMA,
            pltpu.SemaphoreType.DMA,
            pltpu.SemaphoreType.DMA,
            pltpu.SemaphoreType.DMA,
            pltpu.SemaphoreType.DMA,
            pltpu.SemaphoreType.DMA,
        ])(_sc_scatter_body)


def _sc_scatter(pm_flat, t_flat, src, dst):
    return _sc_scatter_kernel()(pm_flat, t_flat, src, dst)


def _sc_scatter_body(pm_hbm, t_hbm, src_hbm, dst_hbm, out_hbm,
                     sv0, sv1, dv0, dv1, dsc0, dsc1,
                     gv0, gv1, tv0, tv1, agg_sh,
                     semi0, semi1, semg0, semg1, semt0, semt1, sems0, sems1):
    c = lax.axis_index("c")
    s = lax.axis_index("s")
    svs, dvs, dscs = (sv0, sv1), (dv0, dv1), (dsc0, dsc1)
    gvs, tvs = (gv0, gv1), (tv0, tv1)
    semi, semg = (semi0, semi1), (semg0, semg1)
    semt, sems = (semt0, semt1), (sems0, sems1)

    def e_at(k):
        return (s + 16 * k) * CHS

    def start_idx(k, b):
        e0 = e_at(k)
        pltpu.async_copy(src_hbm.at[pl.ds(e0, CHS)], svs[b], semi[b])
        pltpu.async_copy(dst_hbm.at[pl.ds(e0, CHS)], dvs[b], semi[b])

    def wait_idx(k, b):
        e0 = e_at(k)
        pltpu.make_async_copy(src_hbm.at[pl.ds(e0, CHS)], svs[b], semi[b]).wait()
        pltpu.make_async_copy(dst_hbm.at[pl.ds(e0, CHS)], dvs[b], semi[b]).wait()

    def wait_scat(b):
        pltpu.make_async_copy(gvs[b], agg_sh.at[dscs[b]], sems[b]).wait()

    def start_gt(k, b):
        e0 = e_at(k)

        pltpu.async_copy(pm_hbm.at[c].at[svs[b]], gvs[b], semg[b])
        pltpu.async_copy(t_hbm.at[c, pl.ds(e0, CHS)], tvs[b], semt[b])

    def finish(k, b):
        e0 = e_at(k)
        pltpu.make_async_copy(pm_hbm.at[c].at[svs[b]], gvs[b], semg[b]).wait()
        pltpu.make_async_copy(
            t_hbm.at[c, pl.ds(e0, CHS)], tvs[b], semt[b]).wait()

        def comp(i, carry):
            for g in range(4):
                v = tvs[b][i, pl.ds(16 * g, 16)]
                bf = plsc.bitcast(v, jnp.bfloat16)
                ta, tb = plsc.unpack(
                    bf, format=plsc.PackFormat.INTERLEAVED)
                sla = pl.ds(32 * g, 16)
                slb = pl.ds(32 * g + 16, 16)
                gvs[b][i, sla] = jnp.maximum(gvs[b][i, sla] + ta, 0.0)
                gvs[b][i, slb] = jnp.maximum(gvs[b][i, slb] + tb, 0.0)
            return carry

        lax.fori_loop(0, CHS, comp, 0, unroll=2)

        def dcp(i, carry):
            sl = pl.ds(i * 16, 16)
            dscs[b][sl] = dvs[b][sl]
            return carry

        lax.fori_loop(0, CHS // 16, dcp, 0, unroll=5)
        pltpu.async_copy(gvs[b], agg_sh.at[dscs[b]], sems[b], add=True)

    start_idx(0, 0)
    start_idx(1, 1)

    def z_body(i, carry):
        for j in range(8):
            gv0[i, pl.ds(j * 16, 16)] = jnp.zeros((16,), jnp.float32)
        return carry

    lax.fori_loop(0, CHS, z_body, 0, unroll=4)
    for r in range(8):
        pltpu.async_copy(gv0, agg_sh.at[pl.ds(s * RPT + r * 80, 80)], semg0)
    for r in range(8):
        pltpu.make_async_copy(
            gv0, agg_sh.at[pl.ds(s * RPT + r * 80, 80)], semg0).wait()
    plsc.subcore_barrier()

    wait_idx(0, 0)
    start_gt(0, 0)

    def body(jj, carry):
        k0 = 2 * jj
        k1 = k0 + 1

        @pl.when(k1 < KPT)
        def _():
            wait_idx(k1, 1)

        @pl.when(jnp.logical_and(k1 < KPT, k1 >= 2))
        def _():
            wait_scat(1)

        @pl.when(k1 < KPT)
        def _():
            start_gt(k1, 1)

        finish(k0, 0)

        @pl.when(k0 + 2 < KPT)
        def _():
            start_idx(k0 + 2, 0)

        @pl.when(k1 < KPT)
        def _():
            finish(k1, 1)

        @pl.when(k0 + 2 < KPT)
        def _():
            wait_idx(k0 + 2, 0)
            wait_scat(0)
            start_gt(k0 + 2, 0)

        @pl.when(k1 + 2 < KPT)
        def _():
            start_idx(k1 + 2, 1)

        return carry

    lax.fori_loop(0, (KPT + 1) // 2, body, 0)
    wait_scat(0)
    wait_scat(1)
    plsc.subcore_barrier()

    for r in range(8):
        b = r % 2
        if r >= 2:
            pltpu.make_async_copy(
                gvs[b],
                out_hbm.at[c, pl.ds(s * RPT + (r - 2) * 80, 80)],
                sems[b]).wait()
        pltpu.sync_copy(agg_sh.at[pl.ds(s * RPT + r * 80, 80)], gvs[b])
        pltpu.async_copy(
            gvs[b], out_hbm.at[c, pl.ds(s * RPT + r * 80, 80)],
            sems[b])
    for r in (6, 7):
        b = r % 2
        pltpu.make_async_copy(
            gvs[b], out_hbm.at[c, pl.ds(s * RPT + r * 80, 80)],
            sems[b]).wait()


def kernel(edge_index, x, z,
           We0, be0, Wm0, bm0, Wn0, bn0,
           We1, be1, Wm1, bm1, Wn1, bn1,
           We2, be2, Wm2, bm2, Wn2, bn2):
    src = edge_index[0].astype(jnp.int32)
    dst = edge_index[1].astype(jnp.int32)
    x = x.astype(jnp.float32)
    ea = z.astype(jnp.float32)

    pes, ped, pm2 = _tc_node_proj(
        x, We0[:DD], We0[DD:2 * DD], Wm0[:DD, :128], Wm0[:DD, 128:])
    gsum = _sc_gather_pe(pes, ped, src, dst)
    ea, t2 = _tc_edge_dense(
        gsum, ea, We0[2 * DD:], be0.reshape(1, DEE),
        Wm0[DD:, :128], Wm0[DD:, 128:],
        bm0[:128].reshape(1, 128), bm0[128:].reshape(1, 128), False)
    agg = _sc_scatter(pm2, t2, src, dst)
    x, pes, ped, pm2 = _tc_node_fused(
        x, agg, Wn0[:DD], Wn0[DD:DD + 128],
        Wn0[DD + 128:], bn0.reshape(1, DD),
        We1[:DD], We1[DD:2 * DD], Wm1[:DD, :128], Wm1[:DD, 128:], False)

    gsum = _sc_gather_pe(pes, ped, src, dst)
    ea, t2 = _tc_edge_dense(
        gsum, ea, We1[2 * DD:], be1.reshape(1, DEE),
        Wm1[DD:, :128], Wm1[DD:, 128:],
        bm1[:128].reshape(1, 128), bm1[128:].reshape(1, 128), True)
    agg = _sc_scatter(pm2, t2, src, dst)
    x, pes, ped = _tc_node_fused(
        x, agg, Wn1[:DD], Wn1[DD:DD + 128],
        Wn1[DD + 128:], bn1.reshape(1, DD),
        We2[:DD], We2[DD:2 * DD], None, None, True)

    gsum = _sc_gather_pe(pes, ped, src, dst)
    return _tc_edge_final(gsum, ea, We2[2 * DD:], be2.reshape(1, DEE))


# speedup vs baseline: 1.2011x; 1.2011x over previous
import functools

import jax
import jax.numpy as jnp
from jax import lax
from jax.experimental import pallas as pl
from jax.experimental.pallas import tpu as pltpu
from jax.experimental.pallas import tpu_sc as plsc

NN = 10000
EE = 160000
DD = 256
DEE = 16
NPAD = 10240
CHG = 128
NCHG = EE // CHG
CHS = 64
NCHS = EE // CHS
KPTB = NCHS // 16
RPT = NPAD // 16

_HI = jax.lax.Precision.HIGHEST


def _node_proj_body(x_ref, wes_ref, wed_ref, wma_ref, wmb_ref,
                    pes_ref, ped_ref, pm_ref):
    x = x_ref[...]
    pes_ref[...] = jnp.dot(x, wes_ref[...])
    ped_ref[...] = jnp.dot(x, wed_ref[...])
    pm_ref[0] = jnp.dot(x, wma_ref[...])
    pm_ref[1] = jnp.dot(x, wmb_ref[...])


def _tc_node_proj(x, wes, wed, wma, wmb):
    bn = 2000
    f = jnp.float32
    return pl.pallas_call(
        _node_proj_body,
        grid=(NN // bn,),
        in_specs=[
            pl.BlockSpec((bn, DD), lambda i: (i, 0)),
            pl.BlockSpec((DD, DEE), lambda i: (0, 0)),
            pl.BlockSpec((DD, DEE), lambda i: (0, 0)),
            pl.BlockSpec((DD, 128), lambda i: (0, 0)),
            pl.BlockSpec((DD, 128), lambda i: (0, 0)),
        ],
        out_specs=[
            pl.BlockSpec((bn, DEE), lambda i: (i, 0)),
            pl.BlockSpec((bn, DEE), lambda i: (i, 0)),
            pl.BlockSpec((2, bn, 128), lambda i: (0, i, 0)),
        ],
        out_shape=[
            jax.ShapeDtypeStruct((NN, DEE), f),
            jax.ShapeDtypeStruct((NN, DEE), f),
            jax.ShapeDtypeStruct((2, NN, 128), f),
        ],
    )(x, wes, wed, wma, wmb)


def _edge_dense_body(gsum_ref, ea_ref, wec_ref, be_ref, wma_ref, wmb_ref,
                     bma_ref, bmb_ref, eout_ref, t_ref, *, res):
    ea = ea_ref[...]
    e = jnp.maximum(
        gsum_ref[...] + jnp.dot(ea, wec_ref[...]) + be_ref[...],
        0.0)
    t_ref[0] = jnp.dot(e, wma_ref[...]) + bma_ref[...]
    t_ref[1] = jnp.dot(e, wmb_ref[...]) + bmb_ref[...]
    eout_ref[...] = 0.5 * (ea + e) if res else e


def _tc_edge_dense(gsum, ea, wec, be, wma, wmb, bma, bmb, res):
    be_ = 3200
    f = jnp.float32
    return pl.pallas_call(
        functools.partial(_edge_dense_body, res=res),
        grid=(EE // be_,),
        in_specs=[
            pl.BlockSpec((be_ // 8, 128), lambda i: (i, 0)),
            pl.BlockSpec((be_ // 8, 128), lambda i: (i, 0)),
            pl.BlockSpec((128, 128), lambda i: (0, 0)),
            pl.BlockSpec((1, 128), lambda i: (0, 0)),
            pl.BlockSpec((128, 1024), lambda i: (0, 0)),
            pl.BlockSpec((128, 1024), lambda i: (0, 0)),
            pl.BlockSpec((1, 1024), lambda i: (0, 0)),
            pl.BlockSpec((1, 1024), lambda i: (0, 0)),
        ],
        out_specs=[
            pl.BlockSpec((be_ // 8, 128), lambda i: (i, 0)),
            pl.BlockSpec((2, be_ // 8, 1024), lambda i: (0, i, 0)),
        ],
        out_shape=[
            jax.ShapeDtypeStruct((EE // 8, 128), f),
            jax.ShapeDtypeStruct((2, EE // 8, 1024), f),
        ],
    )(gsum, ea, wec, be, wma, wmb, bma, bmb)


def _edge_final_body(gsum_ref, ea_ref, wec_ref, be_ref, out_ref):
    out_ref[...] = jnp.maximum(
        gsum_ref[...]
        + jnp.dot(ea_ref[...], wec_ref[...]) + be_ref[...],
        0.0)


def _tc_edge_final(gsum, ea, wec, be):
    be_ = 3200
    return pl.pallas_call(
        _edge_final_body,
        grid=(EE // be_,),
        in_specs=[
            pl.BlockSpec((be_ // 8, 128), lambda i: (i, 0)),
            pl.BlockSpec((be_ // 8, 128), lambda i: (i, 0)),
            pl.BlockSpec((128, 128), lambda i: (0, 0)),
            pl.BlockSpec((1, 128), lambda i: (0, 0)),
        ],
        out_specs=pl.BlockSpec((be_ // 8, 128), lambda i: (i, 0)),
        out_shape=jax.ShapeDtypeStruct((EE // 8, 128), jnp.float32),
    )(gsum, ea, wec, be)


def _node_fused_body(x_ref, agga_ref, aggb_ref, wnt_ref, wna_ref, wnb_ref,
                     bn_ref, wes_ref, wed_ref, *rest, res, has_pm):
    if has_pm:
        wma_ref, wmb_ref, xout_ref, pes_ref, ped_ref, pm_ref = rest
    else:
        xout_ref, pes_ref, ped_ref = rest
    x = x_ref[...]
    h = (jnp.dot(x, wnt_ref[...])
         + jnp.dot(agga_ref[0], wna_ref[...])
         + jnp.dot(aggb_ref[0], wnb_ref[...])
         + bn_ref[...])
    h = jnp.maximum(h, 0.0)
    xn = 0.5 * (x + h) if res else h
    xout_ref[...] = xn
    pes_ref[...] = jnp.dot(xn, wes_ref[...])
    ped_ref[...] = jnp.dot(xn, wed_ref[...])
    if has_pm:
        pm_ref[0] = jnp.dot(xn, wma_ref[...])
        pm_ref[1] = jnp.dot(xn, wmb_ref[...])


def _tc_node_fused(x, agg3, wnt, wna, wnb, bn, wes, wed, wma, wmb, res):
    bn_ = 2000
    f = jnp.float32
    has_pm = wma is not None
    in_specs = [
        pl.BlockSpec((bn_, DD), lambda i: (i, 0)),
        pl.BlockSpec((1, bn_, 128), lambda i: (0, i, 0)),
        pl.BlockSpec((1, bn_, 128), lambda i: (1, i, 0)),
        pl.BlockSpec((DD, DD), lambda i: (0, 0)),
        pl.BlockSpec((128, DD), lambda i: (0, 0)),
        pl.BlockSpec((128, DD), lambda i: (0, 0)),
        pl.BlockSpec((1, DD), lambda i: (0, 0)),
        pl.BlockSpec((DD, DEE), lambda i: (0, 0)),
        pl.BlockSpec((DD, DEE), lambda i: (0, 0)),
    ]
    out_specs = [
        pl.BlockSpec((bn_, DD), lambda i: (i, 0)),
        pl.BlockSpec((bn_, DEE), lambda i: (i, 0)),
        pl.BlockSpec((bn_, DEE), lambda i: (i, 0)),
    ]
    out_shape = [
        jax.ShapeDtypeStruct((NN, DD), f),
        jax.ShapeDtypeStruct((NN, DEE), f),
        jax.ShapeDtypeStruct((NN, DEE), f),
    ]
    args = [x, agg3, agg3, wnt, wna, wnb, bn, wes, wed]
    if has_pm:
        in_specs += [pl.BlockSpec((DD, 128), lambda i: (0, 0)),
                     pl.BlockSpec((DD, 128), lambda i: (0, 0))]
        out_specs.append(pl.BlockSpec((2, bn_, 128), lambda i: (0, i, 0)))
        out_shape.append(jax.ShapeDtypeStruct((2, NN, 128), f))
        args += [wma, wmb]
    return pl.pallas_call(
        functools.partial(_node_fused_body, res=res, has_pm=has_pm),
        grid=(NN // bn_,),
        in_specs=in_specs,
        out_specs=out_specs,
        out_shape=out_shape,
    )(*args)


@functools.cache
def _sc_gather_pe_kernel():
    return functools.partial(
        pl.kernel,
        out_type=jax.ShapeDtypeStruct((EE // 8, 128), jnp.float32),
        mesh=plsc.VectorSubcoreMesh(core_axis_name="c", subcore_axis_name="s"),
        compiler_params=pltpu.CompilerParams(use_tc_tiling_on_sc=False),
        scratch_types=[
            pltpu.VMEM((CHG,), jnp.int32),
            pltpu.VMEM((CHG,), jnp.int32),
            pltpu.VMEM((CHG,), jnp.int32),
            pltpu.VMEM((CHG,), jnp.int32),
            pltpu.VMEM((CHG, DEE), jnp.float32),
            pltpu.VMEM((CHG, DEE), jnp.float32),
            pltpu.VMEM((CHG, DEE), jnp.float32),
            pltpu.VMEM((CHG, DEE), jnp.float32),
            pltpu.VMEM((CHG // 8, 128), jnp.float32),
            pltpu.VMEM((CHG // 8, 128), jnp.float32),
            pltpu.SemaphoreType.DMA,
            pltpu.SemaphoreType.DMA,
            pltpu.SemaphoreType.DMA,
            pltpu.SemaphoreType.DMA,
            pltpu.SemaphoreType.DMA,
            pltpu.SemaphoreType.DMA,
        ])(_sc_gather_pe_body)


def _sc_gather_pe(pes, ped, src, dst):
    return _sc_gather_pe_kernel()(pes, ped, src, dst)


def _sc_gather_pe_body(pes_hbm, ped_hbm, src_hbm, dst_hbm, out_hbm,
                       sv0, sv1, dv0, dv1, gs0, gs1, gd0, gd1, wv0, wv1,
                       semi0, semi1, semg0, semg1, semo0, semo1):
    c = lax.axis_index("c")
    s = lax.axis_index("s")
    w = s * 2 + c
    nk = 39 + jnp.where(w < NCHG - 39 * 32, 1, 0)
    svs, dvs = (sv0, sv1), (dv0, dv1)
    gss, gds = (gs0, gs1), (gd0, gd1)
    wvs = (wv0, wv1)
    semi, semg, semo = (semi0, semi1), (semg0, semg1), (semo0, semo1)

    def e_at(k):
        return (w + 32 * k) * CHG

    def start_idx(k, b):
        e0 = e_at(k)
        pltpu.async_copy(src_hbm.at[pl.ds(e0, CHG)], svs[b], semi[b])
        pltpu.async_copy(dst_hbm.at[pl.ds(e0, CHG)], dvs[b], semi[b])

    def wait_idx(k, b):
        e0 = e_at(k)
        pltpu.make_async_copy(src_hbm.at[pl.ds(e0, CHG)], svs[b], semi[b]).wait()
        pltpu.make_async_copy(dst_hbm.at[pl.ds(e0, CHG)], dvs[b], semi[b]).wait()

    def start_g(k, b):
        pltpu.async_copy(pes_hbm.at[svs[b]], gss[b], semg[b])
        pltpu.async_copy(ped_hbm.at[dvs[b]], gds[b], semg[b])

    def wait_write(k, b):
        e0 = e_at(k)
        pltpu.make_async_copy(
            wvs[b],
            out_hbm.at[pl.ds((w + 32 * k) * (CHG // 8), CHG // 8)],
            semo[b]).wait()

    def finish(k, b):
        e0 = e_at(k)
        pltpu.make_async_copy(pes_hbm.at[svs[b]], gss[b], semg[b]).wait()
        pltpu.make_async_copy(ped_hbm.at[dvs[b]], gds[b], semg[b]).wait()

        def add_body(ii, carry):
            for jc in range(8):
                i = 8 * ii + jc
                wvs[b][ii, pl.ds(16 * jc, 16)] = gss[b][i, :] + gds[b][i, :]
            return carry

        lax.fori_loop(0, CHG // 8, add_body, 0, unroll=2)
        pltpu.async_copy(
            wvs[b], out_hbm.at[pl.ds((w + 32 * k) * (CHG // 8), CHG // 8)],
            semo[b])

    start_idx(0, 0)
    start_idx(1, 1)
    wait_idx(0, 0)
    start_g(0, 0)

    def body(jj, carry):
        k0 = 2 * jj
        k1 = k0 + 1

        @pl.when(k1 < nk)
        def _():
            wait_idx(k1, 1)

        @pl.when(jnp.logical_and(k1 < nk, k1 >= 2))
        def _():
            wait_write(k1 - 2, 1)

        @pl.when(k1 < nk)
        def _():
            start_g(k1, 1)

        finish(k0, 0)

        @pl.when(k0 + 2 < nk)
        def _():
            start_idx(k0 + 2, 0)

        @pl.when(k1 < nk)
        def _():
            finish(k1, 1)

        @pl.when(k0 + 2 < nk)
        def _():
            wait_idx(k0 + 2, 0)
            wait_write(k0, 0)
            start_g(k0 + 2, 0)

        @pl.when(k1 + 2 < nk)
        def _():
            start_idx(k1 + 2, 1)

        return carry

    lax.fori_loop(0, 20, body, 0)
    wait_write(38, 0)
    wait_write(37, 1)


@functools.cache
def _sc_scatter_kernel():
    return functools.partial(
        pl.kernel,
        out_type=jax.ShapeDtypeStruct((2, NPAD, 128), jnp.float32),
        mesh=plsc.VectorSubcoreMesh(core_axis_name="c", subcore_axis_name="s"),
        scratch_types=[
            pltpu.VMEM((CHS,), jnp.int32),
            pltpu.VMEM((CHS,), jnp.int32),
            pltpu.VMEM((CHS,), jnp.int32),
            pltpu.VMEM((CHS,), jnp.int32),
            pltpu.VMEM((CHS,), jnp.int32),
            pltpu.VMEM((CHS,), jnp.int32),
            pltpu.VMEM((CHS,), jnp.int32),
            pltpu.VMEM((CHS,), jnp.int32),
            pltpu.VMEM((CHS, 128), jnp.float32),
            pltpu.VMEM((CHS, 128), jnp.float32),
            pltpu.VMEM((CHS // 8, 1024), jnp.float32),
            pltpu.VMEM((CHS // 8, 1024), jnp.float32),
            pltpu.VMEM_SHARED((NPAD, 128), jnp.float32),
            pltpu.SemaphoreType.DMA,
            pltpu.SemaphoreType.DMA,
            pltpu.SemaphoreType.DMA,
            pltpu.SemaphoreType.DMA,
            pltpu.SemaphoreType.DMA,
            pltpu.SemaphoreType.DMA,
            pltpu.SemaphoreType.DMA,
            pltpu.SemaphoreType.DMA,
        ])(_sc_scatter_body)


def _sc_scatter(pm_flat, t_flat, src, dst):
    return _sc_scatter_kernel()(pm_flat, t_flat, src, dst)


def _sc_scatter_body(pm_hbm, t_hbm, src_hbm, dst_hbm, out_hbm,
                     sv0, sv1, s20, s21, dv0, dv1, dsc0, dsc1,
                     gv0, gv1, tv0, tv1, agg_sh,
                     semi0, semi1, semg0, semg1, semt0, semt1, sems0, sems1):
    c = lax.axis_index("c")
    s = lax.axis_index("s")
    nk = KPTB + jnp.where(s < NCHS - KPTB * 16, 1, 0)
    svs, s2s, dvs, dscs = (sv0, sv1), (s20, s21), (dv0, dv1), (dsc0, dsc1)
    gvs, tvs = (gv0, gv1), (tv0, tv1)
    semi, semg = (semi0, semi1), (semg0, semg1)
    semt, sems = (semt0, semt1), (sems0, sems1)

    def e_at(k):
        return (s + 16 * k) * CHS

    def start_idx(k, b):
        e0 = e_at(k)
        pltpu.async_copy(src_hbm.at[pl.ds(e0, CHS)], svs[b], semi[b])
        pltpu.async_copy(dst_hbm.at[pl.ds(e0, CHS)], dvs[b], semi[b])

    def wait_idx(k, b):
        e0 = e_at(k)
        pltpu.make_async_copy(src_hbm.at[pl.ds(e0, CHS)], svs[b], semi[b]).wait()
        pltpu.make_async_copy(dst_hbm.at[pl.ds(e0, CHS)], dvs[b], semi[b]).wait()

    def wait_scat(b):
        pltpu.make_async_copy(gvs[b], agg_sh.at[dscs[b]], sems[b]).wait()

    def start_gt(k, b):
        e0 = e_at(k)

        pltpu.async_copy(pm_hbm.at[c].at[svs[b]], gvs[b], semg[b])
        pltpu.async_copy(
            t_hbm.at[c, pl.ds((s + 16 * k) * (CHS // 8), CHS // 8)],
            tvs[b], semt[b])

    def finish(k, b):
        e0 = e_at(k)
        pltpu.make_async_copy(pm_hbm.at[c].at[svs[b]], gvs[b], semg[b]).wait()
        pltpu.make_async_copy(
            t_hbm.at[c, pl.ds((s + 16 * k) * (CHS // 8), CHS // 8)],
            tvs[b], semt[b]).wait()

        def comp(rr, carry):
            for h in range(8):
                i = 8 * rr + h
                for jj in range(8):
                    sl = pl.ds(jj * 16, 16)
                    slt = pl.ds(128 * h + 16 * jj, 16)
                    gvs[b][i, sl] = jnp.maximum(
                        gvs[b][i, sl] + tvs[b][rr, slt], 0.0)
            return carry

        lax.fori_loop(0, CHS // 8, comp, 0)

        def dcp(i, carry):
            sl = pl.ds(i * 16, 16)
            dscs[b][sl] = dvs[b][sl]
            return carry

        lax.fori_loop(0, CHS // 16, dcp, 0, unroll=5)
        pltpu.async_copy(gvs[b], agg_sh.at[dscs[b]], sems[b], add=True)

    start_idx(0, 0)
    start_idx(1, 1)

    def z_body(i, carry):
        for j in range(8):
            gv0[i, pl.ds(j * 16, 16)] = jnp.zeros((16,), jnp.float32)
        return carry

    lax.fori_loop(0, CHS, z_body, 0, unroll=4)
    for r in range(RPT // CHS):
        pltpu.async_copy(
            gv0, agg_sh.at[pl.ds(s * RPT + r * CHS, CHS)], semg0)
    for r in range(RPT // CHS):
        pltpu.make_async_copy(
            gv0, agg_sh.at[pl.ds(s * RPT + r * CHS, CHS)], semg0).wait()
    plsc.subcore_barrier()

    wait_idx(0, 0)
    start_gt(0, 0)

    def body(jj, carry):
        k0 = 2 * jj
        k1 = k0 + 1

        @pl.when(k1 < nk)
        def _():
            wait_idx(k1, 1)

        @pl.when(jnp.logical_and(k1 < nk, k1 >= 2))
        def _():
            wait_scat(1)

        @pl.when(k1 < nk)
        def _():
            start_gt(k1, 1)

        @pl.when(k0 < nk)
        def _():
            finish(k0, 0)

        @pl.when(k0 + 2 < nk)
        def _():
            start_idx(k0 + 2, 0)

        @pl.when(k1 < nk)
        def _():
            finish(k1, 1)

        @pl.when(k0 + 2 < nk)
        def _():
            wait_idx(k0 + 2, 0)
            wait_scat(0)
            start_gt(k0 + 2, 0)

        @pl.when(k1 + 2 < nk)
        def _():
            start_idx(k1 + 2, 1)

        return carry

    lax.fori_loop(0, (KPTB + 2) // 2, body, 0)
    wait_scat(0)
    wait_scat(1)
    plsc.subcore_barrier()

    ndr = RPT // CHS
    for r in range(ndr):
        b = r % 2
        if r >= 2:
            pltpu.make_async_copy(
                gvs[b],
                out_hbm.at[c, pl.ds(s * RPT + (r - 2) * CHS, CHS)],
                sems[b]).wait()
        pltpu.sync_copy(agg_sh.at[pl.ds(s * RPT + r * CHS, CHS)], gvs[b])
        pltpu.async_copy(
            gvs[b], out_hbm.at[c, pl.ds(s * RPT + r * CHS, CHS)],
            sems[b])
    for r in (ndr - 2, ndr - 1):
        b = r % 2
        pltpu.make_async_copy(
            gvs[b], out_hbm.at[c, pl.ds(s * RPT + r * CHS, CHS)],
            sems[b]).wait()


def _wide(w):
    return jnp.kron(jnp.eye(8, dtype=w.dtype), w)


def _tile8(b):
    return jnp.tile(b.reshape(1, -1), (1, 8)).reshape(1, -1)


def kernel(edge_index, x, z,
           We0, be0, Wm0, bm0, Wn0, bn0,
           We1, be1, Wm1, bm1, Wn1, bn1,
           We2, be2, Wm2, bm2, Wn2, bn2):
    src = edge_index[0].astype(jnp.int32)
    dst = edge_index[1].astype(jnp.int32)
    x = x.astype(jnp.float32)
    ea = z.astype(jnp.float32).reshape(EE // 8, 8 * DEE)

    pes, ped, pm2 = _tc_node_proj(
        x, We0[:DD], We0[DD:2 * DD], Wm0[:DD, :128], Wm0[:DD, 128:])
    gsum = _sc_gather_pe(pes, ped, src, dst)
    ea, t2 = _tc_edge_dense(
        gsum, ea, _wide(We0[2 * DD:]), _tile8(be0),
        _wide(Wm0[DD:, :128]), _wide(Wm0[DD:, 128:]),
        _tile8(bm0[:128]), _tile8(bm0[128:]), False)
    agg = _sc_scatter(pm2, t2, src, dst)
    x, pes, ped, pm2 = _tc_node_fused(
        x, agg, Wn0[:DD], Wn0[DD:DD + 128],
        Wn0[DD + 128:], bn0.reshape(1, DD),
        We1[:DD], We1[DD:2 * DD], Wm1[:DD, :128], Wm1[:DD, 128:], False)

    gsum = _sc_gather_pe(pes, ped, src, dst)
    ea, t2 = _tc_edge_dense(
        gsum, ea, _wide(We1[2 * DD:]), _tile8(be1),
        _wide(Wm1[DD:, :128]), _wide(Wm1[DD:, 128:]),
        _tile8(bm1[:128]), _tile8(bm1[128:]), True)
    agg = _sc_scatter(pm2, t2, src, dst)
    x, pes, ped = _tc_node_fused(
        x, agg, Wn1[:DD], Wn1[DD:DD + 128],
        Wn1[DD + 128:], bn1.reshape(1, DD),
        We2[:DD], We2[DD:2 * DD], None, None, True)

    gsum = _sc_gather_pe(pes, ped, src, dst)
    out = _tc_edge_final(gsum, ea, _wide(We2[2 * DD:]), _tile8(be2))
    return out.reshape(EE, DEE)
